# Initial kernel scaffold; baseline (speedup 1.0000x reference)
import jax
import jax.numpy as jnp
from jax.experimental import pallas as pl


def kernel(features, edge_index, edgenet_input, pd_ftr_dim, nonimg, m, flag, pae_w1, pae_b1, pae_g, pae_beta, pae_w2, pae_b2, cheb_w0, cheb_w1, cheb_w2, cls_w1, cls_b1, cls_g, cls_beta, cls_w2, cls_b2):
    raise NotImplementedError("write your pallas kernel here")


# speedup vs baseline: 5.6347x; 5.6347x over previous
import functools

import jax
import jax.numpy as jnp
from jax import lax
from jax.experimental import pallas as pl
from jax.experimental.pallas import tpu as pltpu
from jax.experimental.pallas import tpu_sc as plsc

N = 10000
E = 320000
D = 128
NP = 10240
CW = 80
NWORK = 32
RW = E // CW // NWORK
BE = 2000
BNODE = 1000
ZROWS = 400
RB = 25

def _mesh():
    return plsc.VectorSubcoreMesh(core_axis_name="c", subcore_axis_name="s",
                                  num_cores=2, num_subcores=16)


def _pae_sums_body(x1_ref, x2_ref, w1_ref, b1_ref, o_ref):
    @pl.when(pl.program_id(0) == 0)
    def _():
        o_ref[...] = jnp.zeros_like(o_ref)

    h1 = jax.nn.relu(jnp.dot(x1_ref[...], w1_ref[...],
                             preferred_element_type=jnp.float32) + b1_ref[...])
    h2 = jax.nn.relu(jnp.dot(x2_ref[...], w1_ref[...],
                             preferred_element_type=jnp.float32) + b1_ref[...])
    acc = jnp.concatenate([
        jnp.sum(h1, axis=0, keepdims=True),
        jnp.sum(h1 * h1, axis=0, keepdims=True),
        jnp.sum(h2, axis=0, keepdims=True),
        jnp.sum(h2 * h2, axis=0, keepdims=True),
        jnp.zeros((4, 128), jnp.float32),
    ], axis=0)
    o_ref[...] += acc


def _pae_sums(x1, x2, w1, b1r):
    return pl.pallas_call(
        _pae_sums_body,
        grid=(E // BE,),
        in_specs=[
            pl.BlockSpec((BE, 16), lambda i: (i, 0)),
            pl.BlockSpec((BE, 16), lambda i: (i, 0)),
            pl.BlockSpec((16, 128), lambda i: (0, 0)),
            pl.BlockSpec((1, 128), lambda i: (0, 0)),
        ],
        out_specs=pl.BlockSpec((8, 128), lambda i: (0, 0)),
        out_shape=jax.ShapeDtypeStruct((8, 128), jnp.float32),
    )(x1, x2, w1, b1r)


def _pae_ew_body(x1_ref, x2_ref, s_ref, w1_ref, b1_ref, g_ref, be_ref,
                 w2_ref, b2_ref, o_ref):
    inv_e = 1.0 / float(E)

    def branch(x_ref, r0):
        mean = s_ref[r0:r0 + 1, :] * inv_e
        var = s_ref[r0 + 1:r0 + 2, :] * inv_e - mean * mean
        h = jax.nn.relu(jnp.dot(x_ref[...], w1_ref[...],
                                preferred_element_type=jnp.float32) + b1_ref[...])
        hn = (h - mean) * lax.rsqrt(var + 1e-5) * g_ref[...] + be_ref[...]
        return jnp.dot(hn, w2_ref[...],
                       preferred_element_type=jnp.float32) + b2_ref[...]

    g1 = branch(x1_ref, 0)
    g2 = branch(x2_ref, 2)
    n1 = jnp.maximum(jnp.sqrt(jnp.sum(g1 * g1, axis=1, keepdims=True)), 1e-8)
    n2 = jnp.maximum(jnp.sqrt(jnp.sum(g2 * g2, axis=1, keepdims=True)), 1e-8)
    cos = jnp.sum(g1 * g2, axis=1, keepdims=True) / (n1 * n2)
    o_ref[...] = (cos + 1.0) * 0.5


def _pae_ew(x1, x2, sums, w1, b1r, gr, betar, w2, b2r):
    return pl.pallas_call(
        _pae_ew_body,
        grid=(E // BE,),
        in_specs=[
            pl.BlockSpec((BE, 16), lambda i: (i, 0)),
            pl.BlockSpec((BE, 16), lambda i: (i, 0)),
            pl.BlockSpec((8, 128), lambda i: (0, 0)),
            pl.BlockSpec((16, 128), lambda i: (0, 0)),
            pl.BlockSpec((1, 128), lambda i: (0, 0)),
            pl.BlockSpec((1, 128), lambda i: (0, 0)),
            pl.BlockSpec((1, 128), lambda i: (0, 0)),
            pl.BlockSpec((128, 128), lambda i: (0, 0)),
            pl.BlockSpec((1, 128), lambda i: (0, 0)),
        ],
        out_specs=pl.BlockSpec((BE, 1), lambda i: (i, 0)),
        out_shape=jax.ShapeDtypeStruct((E, 1), jnp.float32),
    )(x1, x2, sums, w1, b1r, gr, betar, w2, b2r)


def _dinv_body(dp_ref, o_ref):
    deg = jnp.sum(dp_ref[...], axis=0, keepdims=True)
    dsafe = jnp.where(deg > 0, deg, 1.0)
    o_ref[...] = jnp.where(deg > 0, lax.rsqrt(dsafe), 0.0)


def _tc_dinv(degp):
    return pl.pallas_call(
        _dinv_body,
        out_shape=jax.ShapeDtypeStruct((1, NP), jnp.float32),
    )(degp)


def _addp_body(p_ref, o_ref):
    o_ref[...] = p_ref[0] + p_ref[1]


def _tc_add(parts):
    return pl.pallas_call(
        _addp_body,
        grid=(N // BNODE,),
        in_specs=[pl.BlockSpec((2, BNODE, D), lambda i: (0, i, 0))],
        out_specs=pl.BlockSpec((BNODE, D), lambda i: (i, 0)),
        out_shape=jax.ShapeDtypeStruct((N, D), jnp.float32),
    )(parts)


def _layer_body(y_ref, t1_ref, q_ref, w_ref, o_ref):
    t2 = 2.0 * (q_ref[0] + q_ref[1]) - y_ref[...]
    acc = jnp.dot(y_ref[...], w_ref[0], preferred_element_type=jnp.float32)
    acc += jnp.dot(t1_ref[...], w_ref[1], preferred_element_type=jnp.float32)
    acc += jnp.dot(t2, w_ref[2], preferred_element_type=jnp.float32)
    o_ref[...] = jax.nn.relu(acc)


def _tc_layer(y, t1, qparts, w):
    return pl.pallas_call(
        _layer_body,
        grid=(N // BNODE,),
        in_specs=[
            pl.BlockSpec((BNODE, D), lambda i: (i, 0)),
            pl.BlockSpec((BNODE, D), lambda i: (i, 0)),
            pl.BlockSpec((2, BNODE, D), lambda i: (0, i, 0)),
            pl.BlockSpec((3, D, D), lambda i: (0, 0, 0)),
        ],
        out_specs=pl.BlockSpec((BNODE, D), lambda i: (i, 0)),
        out_shape=jax.ShapeDtypeStruct((N, D), jnp.float32),
    )(y, t1, qparts, w)


def _cls1_body(h_ref, w1_ref, b1_ref, z_ref, s_ref):
    @pl.when(pl.program_id(0) == 0)
    def _():
        s_ref[...] = jnp.zeros_like(s_ref)

    z = jax.nn.relu(jnp.dot(h_ref[...], w1_ref[...],
                            preferred_element_type=jnp.float32) + b1_ref[...])
    z_ref[...] = z
    s_ref[...] += jnp.concatenate([
        jnp.sum(z, axis=0, keepdims=True),
        jnp.sum(z * z, axis=0, keepdims=True),
        jnp.zeros((6, 256), jnp.float32),
    ], axis=0)


def _tc_cls1(h0, w1, b1r):
    return pl.pallas_call(
        _cls1_body,
        grid=(N // BNODE,),
        in_specs=[
            pl.BlockSpec((BNODE, 384), lambda i: (i, 0)),
            pl.BlockSpec((384, 256), lambda i: (0, 0)),
            pl.BlockSpec((1, 256), lambda i: (0, 0)),
        ],
        out_specs=(
            pl.BlockSpec((BNODE, 256), lambda i: (i, 0)),
            pl.BlockSpec((8, 256), lambda i: (0, 0)),
        ),
        out_shape=(
            jax.ShapeDtypeStruct((N, 256), jnp.float32),
            jax.ShapeDtypeStruct((8, 256), jnp.float32),
        ),
    )(h0, w1, b1r)


def _cls2_body(z_ref, s_ref, g_ref, be_ref, w2_ref, b2_ref, o_ref):
    inv_n = 1.0 / float(N)
    mean = s_ref[0:1, :] * inv_n
    var = s_ref[1:2, :] * inv_n - mean * mean
    zn = (z_ref[...] - mean) * lax.rsqrt(var + 1e-5) * g_ref[...] + be_ref[...]
    o_ref[...] = jnp.dot(zn, w2_ref[...],
                         preferred_element_type=jnp.float32) + b2_ref[...]


def _tc_cls2(z, sums, gr, betar, w2p, b2p):
    return pl.pallas_call(
        _cls2_body,
        grid=(N // BNODE,),
        in_specs=[
            pl.BlockSpec((BNODE, 256), lambda i: (i, 0)),
            pl.BlockSpec((8, 256), lambda i: (0, 0)),
            pl.BlockSpec((1, 256), lambda i: (0, 0)),
            pl.BlockSpec((1, 256), lambda i: (0, 0)),
            pl.BlockSpec((256, 128), lambda i: (0, 0)),
            pl.BlockSpec((1, 128), lambda i: (0, 0)),
        ],
        out_specs=pl.BlockSpec((BNODE, 128), lambda i: (i, 0)),
        out_shape=jax.ShapeDtypeStruct((N, 128), jnp.float32),
    )(z, sums, gr, betar, w2p, b2p)


def _sc_deg(src2d, ew2d):
    @functools.partial(
        pl.kernel,
        out_type=jax.ShapeDtypeStruct((NWORK, 1, NP), jnp.float32),
        mesh=_mesh(),
        compiler_params=pltpu.CompilerParams(needs_layout_passes=False),
        scratch_types=[
            pltpu.VMEM((RW, CW), jnp.int32),
            pltpu.VMEM((RW, CW), jnp.float32),
            pltpu.VMEM((NP,), jnp.float32),
        ],
    )
    def k(src_hbm, ew_hbm, out_hbm, src_v, ew_v, acc_v):
        cid = lax.axis_index("c")
        sid = lax.axis_index("s")
        wid = cid * 16 + sid

        def zero(i, _):
            acc_v[pl.ds(i * 16, 16)] = jnp.zeros((16,), jnp.float32)
            return 0
        lax.fori_loop(0, NP // 16, zero, 0)

        pltpu.sync_copy(src_hbm.at[wid], src_v)
        pltpu.sync_copy(ew_hbm.at[wid], ew_v)

        def body(j, _):
            for g in range(CW // 16):
                idx = src_v[j, pl.ds(g * 16, 16)]
                val = ew_v[j, pl.ds(g * 16, 16)]
                plsc.addupdate_scatter(acc_v, [idx], val)
            return 0
        lax.fori_loop(0, RW, body, 0)

        pltpu.sync_copy(acc_v, out_hbm.at[wid, 0])

    return k(src2d, ew2d)


def _sc_nw(dinv1d, src2d, dst2d, ew2d):
    @functools.partial(
        pl.kernel,
        out_type=jax.ShapeDtypeStruct((NWORK, RW, CW), jnp.float32),
        mesh=_mesh(),
        compiler_params=pltpu.CompilerParams(needs_layout_passes=False),
        scratch_types=[
            pltpu.VMEM((NP,), jnp.float32),
            pltpu.VMEM((RW, CW), jnp.int32),
            pltpu.VMEM((RW, CW), jnp.int32),
            pltpu.VMEM((RW, CW), jnp.float32),
            pltpu.VMEM((RW, CW), jnp.float32),
        ],
    )
    def k(dinv_hbm, src_hbm, dst_hbm, ew_hbm, out_hbm,
          dinv_v, src_v, dst_v, ew_v, nw_v):
        cid = lax.axis_index("c")
        sid = lax.axis_index("s")
        wid = cid * 16 + sid

        pltpu.sync_copy(dinv_hbm, dinv_v)
        pltpu.sync_copy(src_hbm.at[wid], src_v)
        pltpu.sync_copy(dst_hbm.at[wid], dst_v)
        pltpu.sync_copy(ew_hbm.at[wid], ew_v)

        def body(j, _):
            for g in range(CW // 16):
                s16 = src_v[j, pl.ds(g * 16, 16)]
                d16 = dst_v[j, pl.ds(g * 16, 16)]
                w16 = ew_v[j, pl.ds(g * 16, 16)]
                ds_ = plsc.load_gather(dinv_v, [s16])
                dd_ = plsc.load_gather(dinv_v, [d16])
                nw_v[j, pl.ds(g * 16, 16)] = -(ds_ * w16 * dd_)
            return 0
        lax.fori_loop(0, RW, body, 0)

        pltpu.sync_copy(nw_v, out_hbm.at[wid])

    return k(dinv1d, src2d, dst2d, ew2d)


def _sc_spmv(y, src4d, dst4d, cf4d):
    @functools.partial(
        pl.kernel,
        out_type=jax.ShapeDtypeStruct((2, N, D), jnp.float32),
        mesh=_mesh(),
        compiler_params=pltpu.CompilerParams(needs_layout_passes=False),
        scratch_types=[
            pltpu.VMEM((RB, CW), jnp.int32),
            pltpu.VMEM((RB, CW), jnp.int32),
            pltpu.VMEM((RB, CW), jnp.float32),
            pltpu.VMEM((CW, D), jnp.float32),
            pltpu.VMEM_SHARED((N, D), jnp.float32),
            pltpu.SemaphoreType.DMA,
        ],
    )
    def k(y_hbm, src_hbm, dst_hbm, cf_hbm, out_hbm,
          src_v, dst_v, cf_v, rows_v, acc_s, sem):
        cid = lax.axis_index("c")
        sid = lax.axis_index("s")
        wid = cid * 16 + sid

        def zrow(i, _):
            for l in range(D // 16):
                rows_v[i, pl.ds(l * 16, 16)] = jnp.zeros((16,), jnp.float32)
            return 0
        lax.fori_loop(0, CW, zrow, 0)

        def zchunk(j, _):
            @pl.when(sid == j % 16)
            def _():
                pltpu.sync_copy(rows_v, acc_s.at[pl.ds(j * CW, CW)])
            return 0
        lax.fori_loop(0, N // CW, zchunk, 0)
        plsc.subcore_barrier()

        def batch(b, _):
            pltpu.sync_copy(src_hbm.at[wid, b], src_v)
            pltpu.sync_copy(dst_hbm.at[wid, b], dst_v)
            pltpu.sync_copy(cf_hbm.at[wid, b], cf_v)

            def chunk(j, _):
                pltpu.async_copy(y_hbm.at[src_v.at[j]], rows_v, sem).wait()

                def scale(g, _):
                    cvec = cf_v[j, pl.ds(g * 16, 16)]
                    for e16 in range(16):
                        c = cvec[e16]
                        e = g * 16 + e16
                        for l in range(D // 16):
                            rows_v[e, pl.ds(l * 16, 16)] = (
                                rows_v[e, pl.ds(l * 16, 16)] * c)
                    return 0
                lax.fori_loop(0, CW // 16, scale, 0)

                pltpu.sync_copy(rows_v, acc_s.at[dst_v.at[j]], add=True)
                return 0
            lax.fori_loop(0, RB, chunk, 0)
            return 0
        lax.fori_loop(0, RW // RB, batch, 0)

        plsc.subcore_barrier()
        for j in range(N // ZROWS):
            @pl.when(sid == j % 16)
            def _():
                pltpu.sync_copy(acc_s.at[pl.ds(j * ZROWS, ZROWS)],
                                out_hbm.at[cid, pl.ds(j * ZROWS, ZROWS)])

    return k(y, src4d, dst4d, cf4d)


def kernel(features, edge_index, edgenet_input, pd_ftr_dim, nonimg, m, flag,
           pae_w1, pae_b1, pae_g, pae_beta, pae_w2, pae_b2,
           cheb_w0, cheb_w1, cheb_w2,
           cls_w1, cls_b1, cls_g, cls_beta, cls_w2, cls_b2):
    x1 = edgenet_input[:, :16]
    x2 = edgenet_input[:, 16:]
    src2d = edge_index[0].astype(jnp.int32).reshape(NWORK, RW, CW)
    dst2d = edge_index[1].astype(jnp.int32).reshape(NWORK, RW, CW)

    b1r = pae_b1.reshape(1, 128)
    gr = pae_g.reshape(1, 128)
    betar = pae_beta.reshape(1, 128)
    b2r = pae_b2.reshape(1, 128)

    sums = _pae_sums(x1, x2, pae_w1, b1r)
    ew = _pae_ew(x1, x2, sums, pae_w1, b1r, gr, betar, pae_w2, b2r)
    ew2d = ew.reshape(NWORK, RW, CW)

    degp = _sc_deg(src2d, ew2d)
    dinv = _tc_dinv(degp.reshape(NWORK, NP))
    nw2d = _sc_nw(dinv.reshape(NP), src2d, dst2d, ew2d)

    src4d = src2d.reshape(NWORK, RW // RB, RB, CW)
    dst4d = dst2d.reshape(NWORK, RW // RB, RB, CW)
    nw4d = nw2d.reshape(NWORK, RW // RB, RB, CW)

    h = features
    hs = []
    for w in (cheb_w0, cheb_w1, cheb_w2):
        aparts = _sc_spmv(h, src4d, dst4d, nw4d)
        t1 = _tc_add(aparts)
        bparts = _sc_spmv(t1, src4d, dst4d, nw4d)
        h = _tc_layer(h, t1, bparts, w)
        hs.append(h)
    h0 = jnp.concatenate(hs, axis=1)

    cb1r = cls_b1.reshape(1, 256)
    cgr = cls_g.reshape(1, 256)
    cbetar = cls_beta.reshape(1, 256)
    w2p = jnp.zeros((256, 128), jnp.float32).at[:, :2].set(cls_w2)
    b2p = jnp.zeros((1, 128), jnp.float32).at[:, :2].set(cls_b2.reshape(1, 2))

    z, csums = _tc_cls1(h0, cls_w1, cb1r)
    logit_pad = _tc_cls2(z, csums, cgr, cbetar, w2p, b2p)
    logit = logit_pad[:, :2]
    return (h0, logit)


# speedup vs baseline: 7.6752x; 1.3621x over previous
import functools

import jax
import jax.numpy as jnp
from jax import lax
from jax.experimental import pallas as pl
from jax.experimental.pallas import tpu as pltpu
from jax.experimental.pallas import tpu_sc as plsc

N = 10000
E = 320000
D = 128
NP = 10240
CW = 80
NWORK = 32
RW = E // CW // NWORK
BE = 2000
BNODE = 1000
ZROWS = 400
RB = 25

def _mesh():
    return plsc.VectorSubcoreMesh(core_axis_name="c", subcore_axis_name="s",
                                  num_cores=2, num_subcores=16)


def _pae_sums_body(x1_ref, x2_ref, w1_ref, b1_ref, o_ref):
    @pl.when(pl.program_id(0) == 0)
    def _():
        o_ref[...] = jnp.zeros_like(o_ref)

    h1 = jax.nn.relu(jnp.dot(x1_ref[...], w1_ref[...],
                             preferred_element_type=jnp.float32) + b1_ref[...])
    h2 = jax.nn.relu(jnp.dot(x2_ref[...], w1_ref[...],
                             preferred_element_type=jnp.float32) + b1_ref[...])
    acc = jnp.concatenate([
        jnp.sum(h1, axis=0, keepdims=True),
        jnp.sum(h1 * h1, axis=0, keepdims=True),
        jnp.sum(h2, axis=0, keepdims=True),
        jnp.sum(h2 * h2, axis=0, keepdims=True),
        jnp.zeros((4, 128), jnp.float32),
    ], axis=0)
    o_ref[...] += acc


def _pae_sums(x1, x2, w1, b1r):
    return pl.pallas_call(
        _pae_sums_body,
        grid=(E // BE,),
        in_specs=[
            pl.BlockSpec((BE, 16), lambda i: (i, 0)),
            pl.BlockSpec((BE, 16), lambda i: (i, 0)),
            pl.BlockSpec((16, 128), lambda i: (0, 0)),
            pl.BlockSpec((1, 128), lambda i: (0, 0)),
        ],
        out_specs=pl.BlockSpec((8, 128), lambda i: (0, 0)),
        out_shape=jax.ShapeDtypeStruct((8, 128), jnp.float32),
    )(x1, x2, w1, b1r)


def _pae_ew_body(x1_ref, x2_ref, s_ref, w1_ref, b1_ref, g_ref, be_ref,
                 w2_ref, b2_ref, o_ref):
    inv_e = 1.0 / float(E)

    def branch(x_ref, r0):
        mean = s_ref[r0:r0 + 1, :] * inv_e
        var = s_ref[r0 + 1:r0 + 2, :] * inv_e - mean * mean
        h = jax.nn.relu(jnp.dot(x_ref[...], w1_ref[...],
                                preferred_element_type=jnp.float32) + b1_ref[...])
        hn = (h - mean) * lax.rsqrt(var + 1e-5) * g_ref[...] + be_ref[...]
        return jnp.dot(hn, w2_ref[...],
                       preferred_element_type=jnp.float32) + b2_ref[...]

    g1 = branch(x1_ref, 0)
    g2 = branch(x2_ref, 2)
    n1 = jnp.maximum(jnp.sqrt(jnp.sum(g1 * g1, axis=1, keepdims=True)), 1e-8)
    n2 = jnp.maximum(jnp.sqrt(jnp.sum(g2 * g2, axis=1, keepdims=True)), 1e-8)
    cos = jnp.sum(g1 * g2, axis=1, keepdims=True) / (n1 * n2)
    o_ref[...] = (cos + 1.0) * 0.5


def _pae_ew(x1, x2, sums, w1, b1r, gr, betar, w2, b2r):
    return pl.pallas_call(
        _pae_ew_body,
        grid=(E // BE,),
        in_specs=[
            pl.BlockSpec((BE, 16), lambda i: (i, 0)),
            pl.BlockSpec((BE, 16), lambda i: (i, 0)),
            pl.BlockSpec((8, 128), lambda i: (0, 0)),
            pl.BlockSpec((16, 128), lambda i: (0, 0)),
            pl.BlockSpec((1, 128), lambda i: (0, 0)),
            pl.BlockSpec((1, 128), lambda i: (0, 0)),
            pl.BlockSpec((1, 128), lambda i: (0, 0)),
            pl.BlockSpec((128, 128), lambda i: (0, 0)),
            pl.BlockSpec((1, 128), lambda i: (0, 0)),
        ],
        out_specs=pl.BlockSpec((BE, 1), lambda i: (i, 0)),
        out_shape=jax.ShapeDtypeStruct((E, 1), jnp.float32),
    )(x1, x2, sums, w1, b1r, gr, betar, w2, b2r)


def _dinv_body(dp_ref, o_ref):
    deg = jnp.sum(dp_ref[...], axis=0, keepdims=True)
    dsafe = jnp.where(deg > 0, deg, 1.0)
    o_ref[...] = jnp.where(deg > 0, lax.rsqrt(dsafe), 0.0)


def _tc_dinv(degp):
    return pl.pallas_call(
        _dinv_body,
        out_shape=jax.ShapeDtypeStruct((1, NP), jnp.float32),
    )(degp)


def _addp_body(p_ref, o_ref):
    o_ref[...] = p_ref[0] + p_ref[1]


def _tc_add(parts):
    return pl.pallas_call(
        _addp_body,
        grid=(N // BNODE,),
        in_specs=[pl.BlockSpec((2, BNODE, D), lambda i: (0, i, 0))],
        out_specs=pl.BlockSpec((BNODE, D), lambda i: (i, 0)),
        out_shape=jax.ShapeDtypeStruct((N, D), jnp.float32),
    )(parts)


def _layer_body(y_ref, t1_ref, q_ref, w_ref, o_ref):
    t2 = 2.0 * (q_ref[0] + q_ref[1]) - y_ref[...]
    acc = jnp.dot(y_ref[...], w_ref[0], preferred_element_type=jnp.float32)
    acc += jnp.dot(t1_ref[...], w_ref[1], preferred_element_type=jnp.float32)
    acc += jnp.dot(t2, w_ref[2], preferred_element_type=jnp.float32)
    o_ref[...] = jax.nn.relu(acc)


def _tc_layer(y, t1, qparts, w):
    return pl.pallas_call(
        _layer_body,
        grid=(N // BNODE,),
        in_specs=[
            pl.BlockSpec((BNODE, D), lambda i: (i, 0)),
            pl.BlockSpec((BNODE, D), lambda i: (i, 0)),
            pl.BlockSpec((2, BNODE, D), lambda i: (0, i, 0)),
            pl.BlockSpec((3, D, D), lambda i: (0, 0, 0)),
        ],
        out_specs=pl.BlockSpec((BNODE, D), lambda i: (i, 0)),
        out_shape=jax.ShapeDtypeStruct((N, D), jnp.float32),
    )(y, t1, qparts, w)


def _cls1_body(h_ref, w1_ref, b1_ref, z_ref, s_ref):
    @pl.when(pl.program_id(0) == 0)
    def _():
        s_ref[...] = jnp.zeros_like(s_ref)

    z = jax.nn.relu(jnp.dot(h_ref[...], w1_ref[...],
                            preferred_element_type=jnp.float32) + b1_ref[...])
    z_ref[...] = z
    s_ref[...] += jnp.concatenate([
        jnp.sum(z, axis=0, keepdims=True),
        jnp.sum(z * z, axis=0, keepdims=True),
        jnp.zeros((6, 256), jnp.float32),
    ], axis=0)


def _tc_cls1(h0, w1, b1r):
    return pl.pallas_call(
        _cls1_body,
        grid=(N // BNODE,),
        in_specs=[
            pl.BlockSpec((BNODE, 384), lambda i: (i, 0)),
            pl.BlockSpec((384, 256), lambda i: (0, 0)),
            pl.BlockSpec((1, 256), lambda i: (0, 0)),
        ],
        out_specs=(
            pl.BlockSpec((BNODE, 256), lambda i: (i, 0)),
            pl.BlockSpec((8, 256), lambda i: (0, 0)),
        ),
        out_shape=(
            jax.ShapeDtypeStruct((N, 256), jnp.float32),
            jax.ShapeDtypeStruct((8, 256), jnp.float32),
        ),
    )(h0, w1, b1r)


def _cls2_body(z_ref, s_ref, g_ref, be_ref, w2_ref, b2_ref, o_ref):
    inv_n = 1.0 / float(N)
    mean = s_ref[0:1, :] * inv_n
    var = s_ref[1:2, :] * inv_n - mean * mean
    zn = (z_ref[...] - mean) * lax.rsqrt(var + 1e-5) * g_ref[...] + be_ref[...]
    o_ref[...] = jnp.dot(zn, w2_ref[...],
                         preferred_element_type=jnp.float32) + b2_ref[...]


def _tc_cls2(z, sums, gr, betar, w2p, b2p):
    return pl.pallas_call(
        _cls2_body,
        grid=(N // BNODE,),
        in_specs=[
            pl.BlockSpec((BNODE, 256), lambda i: (i, 0)),
            pl.BlockSpec((8, 256), lambda i: (0, 0)),
            pl.BlockSpec((1, 256), lambda i: (0, 0)),
            pl.BlockSpec((1, 256), lambda i: (0, 0)),
            pl.BlockSpec((256, 128), lambda i: (0, 0)),
            pl.BlockSpec((1, 128), lambda i: (0, 0)),
        ],
        out_specs=pl.BlockSpec((BNODE, 128), lambda i: (i, 0)),
        out_shape=jax.ShapeDtypeStruct((N, 128), jnp.float32),
    )(z, sums, gr, betar, w2p, b2p)


def _sc_deg(src2d, ew2d):
    @functools.partial(
        pl.kernel,
        out_type=jax.ShapeDtypeStruct((NWORK, 1, NP), jnp.float32),
        mesh=_mesh(),
        compiler_params=pltpu.CompilerParams(needs_layout_passes=False),
        scratch_types=[
            pltpu.VMEM((RW, CW), jnp.int32),
            pltpu.VMEM((RW, CW), jnp.float32),
            pltpu.VMEM((NP,), jnp.float32),
        ],
    )
    def k(src_hbm, ew_hbm, out_hbm, src_v, ew_v, acc_v):
        cid = lax.axis_index("c")
        sid = lax.axis_index("s")
        wid = cid * 16 + sid

        def zero(i, _):
            acc_v[pl.ds(i * 16, 16)] = jnp.zeros((16,), jnp.float32)
            return 0
        lax.fori_loop(0, NP // 16, zero, 0)

        pltpu.sync_copy(src_hbm.at[wid], src_v)
        pltpu.sync_copy(ew_hbm.at[wid], ew_v)

        def body(j, _):
            for g in range(CW // 16):
                idx = src_v[j, pl.ds(g * 16, 16)]
                val = ew_v[j, pl.ds(g * 16, 16)]
                plsc.addupdate_scatter(acc_v, [idx], val)
            return 0
        lax.fori_loop(0, RW, body, 0)

        pltpu.sync_copy(acc_v, out_hbm.at[wid, 0])

    return k(src2d, ew2d)


def _sc_nw(dinv1d, src2d, dst2d, ew2d):
    @functools.partial(
        pl.kernel,
        out_type=jax.ShapeDtypeStruct((NWORK, RW, CW), jnp.float32),
        mesh=_mesh(),
        compiler_params=pltpu.CompilerParams(needs_layout_passes=False),
        scratch_types=[
            pltpu.VMEM((NP,), jnp.float32),
            pltpu.VMEM((RW, CW), jnp.int32),
            pltpu.VMEM((RW, CW), jnp.int32),
            pltpu.VMEM((RW, CW), jnp.float32),
            pltpu.VMEM((RW, CW), jnp.float32),
        ],
    )
    def k(dinv_hbm, src_hbm, dst_hbm, ew_hbm, out_hbm,
          dinv_v, src_v, dst_v, ew_v, nw_v):
        cid = lax.axis_index("c")
        sid = lax.axis_index("s")
        wid = cid * 16 + sid

        pltpu.sync_copy(dinv_hbm, dinv_v)
        pltpu.sync_copy(src_hbm.at[wid], src_v)
        pltpu.sync_copy(dst_hbm.at[wid], dst_v)
        pltpu.sync_copy(ew_hbm.at[wid], ew_v)

        def body(j, _):
            for g in range(CW // 16):
                s16 = src_v[j, pl.ds(g * 16, 16)]
                d16 = dst_v[j, pl.ds(g * 16, 16)]
                w16 = ew_v[j, pl.ds(g * 16, 16)]
                ds_ = plsc.load_gather(dinv_v, [s16])
                dd_ = plsc.load_gather(dinv_v, [d16])
                nw_v[j, pl.ds(g * 16, 16)] = -(ds_ * w16 * dd_)
            return 0
        lax.fori_loop(0, RW, body, 0)

        pltpu.sync_copy(nw_v, out_hbm.at[wid])

    return k(dinv1d, src2d, dst2d, ew2d)


def _sc_spmv(y, src4d, dst4d, cf4d):
    @functools.partial(
        pl.kernel,
        out_type=jax.ShapeDtypeStruct((2, N, D), jnp.float32),
        mesh=_mesh(),
        compiler_params=pltpu.CompilerParams(needs_layout_passes=False),
        scratch_types=[
            pltpu.VMEM((RB, CW), jnp.int32),
            pltpu.VMEM((RB, CW), jnp.int32),
            pltpu.VMEM((RB, CW), jnp.float32),
            pltpu.VMEM((CW, D), jnp.float32),
            pltpu.VMEM((CW, D), jnp.float32),
            pltpu.VMEM_SHARED((N, D), jnp.float32),
            pltpu.SemaphoreType.DMA,
            pltpu.SemaphoreType.DMA,
            pltpu.SemaphoreType.DMA,
            pltpu.SemaphoreType.DMA,
        ],
    )
    def k(y_hbm, src_hbm, dst_hbm, cf_hbm, out_hbm,
          src_v, dst_v, cf_v, rows_a, rows_b, acc_s, gsa, gsb, ssa, ssb):
        cid = lax.axis_index("c")
        sid = lax.axis_index("s")
        wid = cid * 16 + sid

        def zrow(i, _):
            for l in range(D // 16):
                rows_a[i, pl.ds(l * 16, 16)] = jnp.zeros((16,), jnp.float32)
            return 0
        lax.fori_loop(0, CW, zrow, 0)

        def zchunk(j, _):
            @pl.when(sid == j % 16)
            def _():
                pltpu.sync_copy(rows_a, acc_s.at[pl.ds(j * CW, CW)])
            return 0
        lax.fori_loop(0, N // CW, zchunk, 0)
        plsc.subcore_barrier()

        def scale_rows(rows_x, jrow):
            def scale(g, _):
                cvec = cf_v[jrow, pl.ds(g * 16, 16)]
                for e16 in range(16):
                    c = cvec[e16]
                    e = g * 16 + e16
                    for l in range(D // 16):
                        rows_x[e, pl.ds(l * 16, 16)] = (
                            rows_x[e, pl.ds(l * 16, 16)] * c)
                return 0
            lax.fori_loop(0, CW // 16, scale, 0)

        def gwait(rows_x, jrow, sem_x):
            pltpu.make_async_copy(y_hbm.at[src_v.at[jrow]], rows_x, sem_x).wait()

        def swait(rows_x, jrow, sem_x):
            pltpu.make_async_copy(rows_x, acc_s.at[dst_v.at[jrow]], sem_x).wait()

        def batch(b, _):
            @pl.when(b >= 1)
            def _():
                swait(rows_a, RB - 1, ssa)
                swait(rows_b, RB - 2, ssb)
            pltpu.sync_copy(src_hbm.at[wid, b], src_v)
            pltpu.sync_copy(dst_hbm.at[wid, b], dst_v)
            pltpu.sync_copy(cf_hbm.at[wid, b], cf_v)
            pltpu.async_copy(y_hbm.at[src_v.at[0]], rows_a, gsa)

            def pair(m2, _):
                j0 = 2 * m2
                j1 = j0 + 1
                @pl.when(m2 >= 1)
                def _():
                    swait(rows_b, j0 - 1, ssb)
                pltpu.async_copy(y_hbm.at[src_v.at[j1]], rows_b, gsb)
                gwait(rows_a, j0, gsa)
                scale_rows(rows_a, j0)
                pltpu.async_copy(rows_a, acc_s.at[dst_v.at[j0]], ssa, add=True)
                swait(rows_a, j0, ssa)
                pltpu.async_copy(y_hbm.at[src_v.at[j1 + 1]], rows_a, gsa)
                gwait(rows_b, j1, gsb)
                scale_rows(rows_b, j1)
                pltpu.async_copy(rows_b, acc_s.at[dst_v.at[j1]], ssb, add=True)
                return 0
            lax.fori_loop(0, RB // 2, pair, 0)
            gwait(rows_a, RB - 1, gsa)
            scale_rows(rows_a, RB - 1)
            pltpu.async_copy(rows_a, acc_s.at[dst_v.at[RB - 1]], ssa, add=True)
            return 0
        lax.fori_loop(0, RW // RB, batch, 0)

        swait(rows_a, RB - 1, ssa)
        swait(rows_b, RB - 2, ssb)
        plsc.subcore_barrier()
        for j in range(N // ZROWS):
            @pl.when(sid == j % 16)
            def _():
                pltpu.sync_copy(acc_s.at[pl.ds(j * ZROWS, ZROWS)],
                                out_hbm.at[cid, pl.ds(j * ZROWS, ZROWS)])

    return k(y, src4d, dst4d, cf4d)


def kernel(features, edge_index, edgenet_input, pd_ftr_dim, nonimg, m, flag,
           pae_w1, pae_b1, pae_g, pae_beta, pae_w2, pae_b2,
           cheb_w0, cheb_w1, cheb_w2,
           cls_w1, cls_b1, cls_g, cls_beta, cls_w2, cls_b2):
    x1 = edgenet_input[:, :16]
    x2 = edgenet_input[:, 16:]
    src2d = edge_index[0].astype(jnp.int32).reshape(NWORK, RW, CW)
    dst2d = edge_index[1].astype(jnp.int32).reshape(NWORK, RW, CW)

    b1r = pae_b1.reshape(1, 128)
    gr = pae_g.reshape(1, 128)
    betar = pae_beta.reshape(1, 128)
    b2r = pae_b2.reshape(1, 128)

    sums = _pae_sums(x1, x2, pae_w1, b1r)
    ew = _pae_ew(x1, x2, sums, pae_w1, b1r, gr, betar, pae_w2, b2r)
    ew2d = ew.reshape(NWORK, RW, CW)

    degp = _sc_deg(src2d, ew2d)
    dinv = _tc_dinv(degp.reshape(NWORK, NP))
    nw2d = _sc_nw(dinv.reshape(NP), src2d, dst2d, ew2d)

    src4d = src2d.reshape(NWORK, RW // RB, RB, CW)
    dst4d = dst2d.reshape(NWORK, RW // RB, RB, CW)
    nw4d = nw2d.reshape(NWORK, RW // RB, RB, CW)

    h = features
    hs = []
    for w in (cheb_w0, cheb_w1, cheb_w2):
        aparts = _sc_spmv(h, src4d, dst4d, nw4d)
        t1 = _tc_add(aparts)
        bparts = _sc_spmv(t1, src4d, dst4d, nw4d)
        h = _tc_layer(h, t1, bparts, w)
        hs.append(h)
    h0 = jnp.concatenate(hs, axis=1)

    cb1r = cls_b1.reshape(1, 256)
    cgr = cls_g.reshape(1, 256)
    cbetar = cls_beta.reshape(1, 256)
    w2p = jnp.zeros((256, 128), jnp.float32).at[:, :2].set(cls_w2)
    b2p = jnp.zeros((1, 128), jnp.float32).at[:, :2].set(cls_b2.reshape(1, 2))

    z, csums = _tc_cls1(h0, cls_w1, cb1r)
    logit_pad = _tc_cls2(z, csums, cgr, cbetar, w2p, b2p)
    logit = logit_pad[:, :2]
    return (h0, logit)


# speedup vs baseline: 8.1546x; 1.0625x over previous
import functools

import jax
import jax.numpy as jnp
from jax import lax
from jax.experimental import pallas as pl
from jax.experimental.pallas import tpu as pltpu
from jax.experimental.pallas import tpu_sc as plsc

N = 10000
E = 320000
D = 128
NP = 10240
CW = 80
NWORK = 32
RW = E // CW // NWORK
BE = 2000
BNODE = 1000
ZROWS = 400
RB = 25

def _mesh():
    return plsc.VectorSubcoreMesh(core_axis_name="c", subcore_axis_name="s",
                                  num_cores=2, num_subcores=16)


def _pae_sums_body(xe_ref, w1_ref, b1_ref, o_ref):
    @pl.when(pl.program_id(0) == 0)
    def _():
        o_ref[...] = jnp.zeros_like(o_ref)

    xe = xe_ref[...].astype(jnp.bfloat16)
    w1b = w1_ref[...].astype(jnp.bfloat16)
    h1 = jax.nn.relu(jnp.dot(xe[:, :16], w1b,
                             preferred_element_type=jnp.float32) + b1_ref[...])
    h2 = jax.nn.relu(jnp.dot(xe[:, 16:], w1b,
                             preferred_element_type=jnp.float32) + b1_ref[...])
    ones = jnp.ones((8, BE), jnp.bfloat16)
    h1b = h1.astype(jnp.bfloat16)
    h2b = h2.astype(jnp.bfloat16)
    s1 = jnp.dot(ones, h1b, preferred_element_type=jnp.float32)
    q1 = jnp.dot(ones, h1b * h1b, preferred_element_type=jnp.float32)
    s2 = jnp.dot(ones, h2b, preferred_element_type=jnp.float32)
    q2 = jnp.dot(ones, h2b * h2b, preferred_element_type=jnp.float32)
    acc = jnp.concatenate([s1[0:1], q1[0:1], s2[0:1], q2[0:1],
                           jnp.zeros((4, 128), jnp.float32)], axis=0)
    o_ref[...] += acc


def _pae_sums(xe, w1, b1r):
    return pl.pallas_call(
        _pae_sums_body,
        grid=(E // BE,),
        in_specs=[
            pl.BlockSpec((BE, 32), lambda i: (i, 0)),
            pl.BlockSpec((16, 128), lambda i: (0, 0)),
            pl.BlockSpec((1, 128), lambda i: (0, 0)),
        ],
        out_specs=pl.BlockSpec((8, 128), lambda i: (0, 0)),
        out_shape=jax.ShapeDtypeStruct((8, 128), jnp.float32),
    )(xe, w1, b1r)


def _pae_ew_body(xe_ref, s_ref, w1_ref, b1_ref, g_ref, be_ref,
                 w2_ref, b2_ref, o_ref):
    inv_e = 1.0 / float(E)
    xe = xe_ref[...].astype(jnp.bfloat16)
    w1b = w1_ref[...].astype(jnp.bfloat16)
    w2b = w2_ref[...].astype(jnp.bfloat16)

    def branch(x, r0):
        mean = s_ref[r0:r0 + 1, :] * inv_e
        var = s_ref[r0 + 1:r0 + 2, :] * inv_e - mean * mean
        h = jax.nn.relu(jnp.dot(x, w1b,
                                preferred_element_type=jnp.float32) + b1_ref[...])
        hn = (h - mean) * lax.rsqrt(var + 1e-5) * g_ref[...] + be_ref[...]
        return jnp.dot(hn.astype(jnp.bfloat16), w2b,
                       preferred_element_type=jnp.float32) + b2_ref[...]

    g1 = branch(xe[:, :16], 0)
    g2 = branch(xe[:, 16:], 2)
    n1 = jnp.maximum(jnp.sqrt(jnp.sum(g1 * g1, axis=1, keepdims=True)), 1e-8)
    n2 = jnp.maximum(jnp.sqrt(jnp.sum(g2 * g2, axis=1, keepdims=True)), 1e-8)
    cos = jnp.sum(g1 * g2, axis=1, keepdims=True) / (n1 * n2)
    o_ref[...] = (cos + 1.0) * 0.5


def _pae_ew(xe, sums, w1, b1r, gr, betar, w2, b2r):
    return pl.pallas_call(
        _pae_ew_body,
        grid=(E // BE,),
        in_specs=[
            pl.BlockSpec((BE, 32), lambda i: (i, 0)),
            pl.BlockSpec((8, 128), lambda i: (0, 0)),
            pl.BlockSpec((16, 128), lambda i: (0, 0)),
            pl.BlockSpec((1, 128), lambda i: (0, 0)),
            pl.BlockSpec((1, 128), lambda i: (0, 0)),
            pl.BlockSpec((1, 128), lambda i: (0, 0)),
            pl.BlockSpec((128, 128), lambda i: (0, 0)),
            pl.BlockSpec((1, 128), lambda i: (0, 0)),
        ],
        out_specs=pl.BlockSpec((BE, 1), lambda i: (i, 0)),
        out_shape=jax.ShapeDtypeStruct((E, 1), jnp.float32),
    )(xe, sums, w1, b1r, gr, betar, w2, b2r)


def _dinv_body(dp_ref, o_ref):
    deg = jnp.sum(dp_ref[...], axis=0, keepdims=True)
    dsafe = jnp.where(deg > 0, deg, 1.0)
    o_ref[...] = jnp.where(deg > 0, lax.rsqrt(dsafe), 0.0)


def _tc_dinv(degp):
    return pl.pallas_call(
        _dinv_body,
        out_shape=jax.ShapeDtypeStruct((1, NP), jnp.float32),
    )(degp)


def _addp_body(p_ref, o_ref):
    o_ref[...] = p_ref[0] + p_ref[1]


def _tc_add(parts):
    return pl.pallas_call(
        _addp_body,
        grid=(N // BNODE,),
        in_specs=[pl.BlockSpec((2, BNODE, D), lambda i: (0, i, 0))],
        out_specs=pl.BlockSpec((BNODE, D), lambda i: (i, 0)),
        out_shape=jax.ShapeDtypeStruct((N, D), jnp.float32),
    )(parts)


def _layer_body(y_ref, t1_ref, q_ref, w_ref, o_ref):
    t2 = 2.0 * (q_ref[0] + q_ref[1]) - y_ref[...]
    acc = jnp.dot(y_ref[...], w_ref[0], preferred_element_type=jnp.float32)
    acc += jnp.dot(t1_ref[...], w_ref[1], preferred_element_type=jnp.float32)
    acc += jnp.dot(t2, w_ref[2], preferred_element_type=jnp.float32)
    o_ref[...] = jax.nn.relu(acc)


def _tc_layer(y, t1, qparts, w):
    return pl.pallas_call(
        _layer_body,
        grid=(N // BNODE,),
        in_specs=[
            pl.BlockSpec((BNODE, D), lambda i: (i, 0)),
            pl.BlockSpec((BNODE, D), lambda i: (i, 0)),
            pl.BlockSpec((2, BNODE, D), lambda i: (0, i, 0)),
            pl.BlockSpec((3, D, D), lambda i: (0, 0, 0)),
        ],
        out_specs=pl.BlockSpec((BNODE, D), lambda i: (i, 0)),
        out_shape=jax.ShapeDtypeStruct((N, D), jnp.float32),
    )(y, t1, qparts, w)


def _cls1_body(h_ref, w1_ref, b1_ref, z_ref, s_ref):
    @pl.when(pl.program_id(0) == 0)
    def _():
        s_ref[...] = jnp.zeros_like(s_ref)

    z = jax.nn.relu(jnp.dot(h_ref[...], w1_ref[...],
                            preferred_element_type=jnp.float32) + b1_ref[...])
    z_ref[...] = z
    s_ref[...] += jnp.concatenate([
        jnp.sum(z, axis=0, keepdims=True),
        jnp.sum(z * z, axis=0, keepdims=True),
        jnp.zeros((6, 256), jnp.float32),
    ], axis=0)


def _tc_cls1(h0, w1, b1r):
    return pl.pallas_call(
        _cls1_body,
        grid=(N // BNODE,),
        in_specs=[
            pl.BlockSpec((BNODE, 384), lambda i: (i, 0)),
            pl.BlockSpec((384, 256), lambda i: (0, 0)),
            pl.BlockSpec((1, 256), lambda i: (0, 0)),
        ],
        out_specs=(
            pl.BlockSpec((BNODE, 256), lambda i: (i, 0)),
            pl.BlockSpec((8, 256), lambda i: (0, 0)),
        ),
        out_shape=(
            jax.ShapeDtypeStruct((N, 256), jnp.float32),
            jax.ShapeDtypeStruct((8, 256), jnp.float32),
        ),
    )(h0, w1, b1r)


def _cls2_body(z_ref, s_ref, g_ref, be_ref, w2_ref, b2_ref, o_ref):
    inv_n = 1.0 / float(N)
    mean = s_ref[0:1, :] * inv_n
    var = s_ref[1:2, :] * inv_n - mean * mean
    zn = (z_ref[...] - mean) * lax.rsqrt(var + 1e-5) * g_ref[...] + be_ref[...]
    o_ref[...] = jnp.dot(zn, w2_ref[...],
                         preferred_element_type=jnp.float32) + b2_ref[...]


def _tc_cls2(z, sums, gr, betar, w2p, b2p):
    return pl.pallas_call(
        _cls2_body,
        grid=(N // BNODE,),
        in_specs=[
            pl.BlockSpec((BNODE, 256), lambda i: (i, 0)),
            pl.BlockSpec((8, 256), lambda i: (0, 0)),
            pl.BlockSpec((1, 256), lambda i: (0, 0)),
            pl.BlockSpec((1, 256), lambda i: (0, 0)),
            pl.BlockSpec((256, 128), lambda i: (0, 0)),
            pl.BlockSpec((1, 128), lambda i: (0, 0)),
        ],
        out_specs=pl.BlockSpec((BNODE, 128), lambda i: (i, 0)),
        out_shape=jax.ShapeDtypeStruct((N, 128), jnp.float32),
    )(z, sums, gr, betar, w2p, b2p)


def _sc_deg(src2d, ew2d):
    @functools.partial(
        pl.kernel,
        out_type=jax.ShapeDtypeStruct((NWORK, 1, NP), jnp.float32),
        mesh=_mesh(),
        compiler_params=pltpu.CompilerParams(needs_layout_passes=False),
        scratch_types=[
            pltpu.VMEM((RW, CW), jnp.int32),
            pltpu.VMEM((RW, CW), jnp.float32),
            pltpu.VMEM((NP,), jnp.float32),
        ],
    )
    def k(src_hbm, ew_hbm, out_hbm, src_v, ew_v, acc_v):
        cid = lax.axis_index("c")
        sid = lax.axis_index("s")
        wid = cid * 16 + sid

        def zero(i, _):
            acc_v[pl.ds(i * 16, 16)] = jnp.zeros((16,), jnp.float32)
            return 0
        lax.fori_loop(0, NP // 16, zero, 0)

        pltpu.sync_copy(src_hbm.at[wid], src_v)
        pltpu.sync_copy(ew_hbm.at[wid], ew_v)

        def body(j, _):
            for g in range(CW // 16):
                idx = src_v[j, pl.ds(g * 16, 16)]
                val = ew_v[j, pl.ds(g * 16, 16)]
                plsc.addupdate_scatter(acc_v, [idx], val)
            return 0
        lax.fori_loop(0, RW, body, 0)

        pltpu.sync_copy(acc_v, out_hbm.at[wid, 0])

    return k(src2d, ew2d)


def _sc_nw(dinv1d, src2d, dst2d, ew2d):
    @functools.partial(
        pl.kernel,
        out_type=jax.ShapeDtypeStruct((NWORK, RW, CW), jnp.float32),
        mesh=_mesh(),
        compiler_params=pltpu.CompilerParams(needs_layout_passes=False),
        scratch_types=[
            pltpu.VMEM((NP,), jnp.float32),
            pltpu.VMEM((RW, CW), jnp.int32),
            pltpu.VMEM((RW, CW), jnp.int32),
            pltpu.VMEM((RW, CW), jnp.float32),
            pltpu.VMEM((RW, CW), jnp.float32),
        ],
    )
    def k(dinv_hbm, src_hbm, dst_hbm, ew_hbm, out_hbm,
          dinv_v, src_v, dst_v, ew_v, nw_v):
        cid = lax.axis_index("c")
        sid = lax.axis_index("s")
        wid = cid * 16 + sid

        pltpu.sync_copy(dinv_hbm, dinv_v)
        pltpu.sync_copy(src_hbm.at[wid], src_v)
        pltpu.sync_copy(dst_hbm.at[wid], dst_v)
        pltpu.sync_copy(ew_hbm.at[wid], ew_v)

        def body(j, _):
            for g in range(CW // 16):
                s16 = src_v[j, pl.ds(g * 16, 16)]
                d16 = dst_v[j, pl.ds(g * 16, 16)]
                w16 = ew_v[j, pl.ds(g * 16, 16)]
                ds_ = plsc.load_gather(dinv_v, [s16])
                dd_ = plsc.load_gather(dinv_v, [d16])
                nw_v[j, pl.ds(g * 16, 16)] = -(ds_ * w16 * dd_)
            return 0
        lax.fori_loop(0, RW, body, 0)

        pltpu.sync_copy(nw_v, out_hbm.at[wid])

    return k(dinv1d, src2d, dst2d, ew2d)


def _sc_spmv(y, src4d, dst4d, cf4d):
    @functools.partial(
        pl.kernel,
        out_type=jax.ShapeDtypeStruct((2, N, D), jnp.float32),
        mesh=_mesh(),
        compiler_params=pltpu.CompilerParams(needs_layout_passes=False),
        scratch_types=[
            pltpu.VMEM((RB, CW), jnp.int32),
            pltpu.VMEM((RB, CW), jnp.int32),
            pltpu.VMEM((RB, CW), jnp.float32),
            pltpu.VMEM((CW, D), jnp.float32),
            pltpu.VMEM((CW, D), jnp.float32),
            pltpu.VMEM_SHARED((N, D), jnp.float32),
            pltpu.SemaphoreType.DMA,
            pltpu.SemaphoreType.DMA,
            pltpu.SemaphoreType.DMA,
            pltpu.SemaphoreType.DMA,
        ],
    )
    def k(y_hbm, src_hbm, dst_hbm, cf_hbm, out_hbm,
          src_v, dst_v, cf_v, rows_a, rows_b, acc_s, gsa, gsb, ssa, ssb):
        cid = lax.axis_index("c")
        sid = lax.axis_index("s")
        wid = cid * 16 + sid

        def zrow(i, _):
            for l in range(D // 16):
                rows_a[i, pl.ds(l * 16, 16)] = jnp.zeros((16,), jnp.float32)
            return 0
        lax.fori_loop(0, CW, zrow, 0)

        def zchunk(j, _):
            @pl.when(sid == j % 16)
            def _():
                pltpu.sync_copy(rows_a, acc_s.at[pl.ds(j * CW, CW)])
            return 0
        lax.fori_loop(0, N // CW, zchunk, 0)
        plsc.subcore_barrier()

        def scale_rows(rows_x, jrow):
            def scale(g, _):
                cvec = cf_v[jrow, pl.ds(g * 16, 16)]
                for e16 in range(16):
                    c = cvec[e16]
                    e = g * 16 + e16
                    for l in range(D // 16):
                        rows_x[e, pl.ds(l * 16, 16)] = (
                            rows_x[e, pl.ds(l * 16, 16)] * c)
                return 0
            lax.fori_loop(0, CW // 16, scale, 0)

        def gwait(rows_x, jrow, sem_x):
            pltpu.make_async_copy(y_hbm.at[src_v.at[jrow]], rows_x, sem_x).wait()

        def swait(rows_x, jrow, sem_x):
            pltpu.make_async_copy(rows_x, acc_s.at[dst_v.at[jrow]], sem_x).wait()

        def batch(b, _):
            @pl.when(b >= 1)
            def _():
                swait(rows_a, RB - 1, ssa)
                swait(rows_b, RB - 2, ssb)
            pltpu.sync_copy(src_hbm.at[wid, b], src_v)
            pltpu.sync_copy(dst_hbm.at[wid, b], dst_v)
            pltpu.sync_copy(cf_hbm.at[wid, b], cf_v)
            pltpu.async_copy(y_hbm.at[src_v.at[0]], rows_a, gsa)

            def pair(m2, _):
                j0 = 2 * m2
                j1 = j0 + 1
                @pl.when(m2 >= 1)
                def _():
                    swait(rows_b, j0 - 1, ssb)
                pltpu.async_copy(y_hbm.at[src_v.at[j1]], rows_b, gsb)
                gwait(rows_a, j0, gsa)
                scale_rows(rows_a, j0)
                pltpu.async_copy(rows_a, acc_s.at[dst_v.at[j0]], ssa, add=True)
                swait(rows_a, j0, ssa)
                pltpu.async_copy(y_hbm.at[src_v.at[j1 + 1]], rows_a, gsa)
                gwait(rows_b, j1, gsb)
                scale_rows(rows_b, j1)
                pltpu.async_copy(rows_b, acc_s.at[dst_v.at[j1]], ssb, add=True)
                return 0
            lax.fori_loop(0, RB // 2, pair, 0)
            gwait(rows_a, RB - 1, gsa)
            scale_rows(rows_a, RB - 1)
            pltpu.async_copy(rows_a, acc_s.at[dst_v.at[RB - 1]], ssa, add=True)
            return 0
        lax.fori_loop(0, RW // RB, batch, 0)

        swait(rows_a, RB - 1, ssa)
        swait(rows_b, RB - 2, ssb)
        plsc.subcore_barrier()
        for j in range(N // ZROWS):
            @pl.when(sid == j % 16)
            def _():
                pltpu.sync_copy(acc_s.at[pl.ds(j * ZROWS, ZROWS)],
                                out_hbm.at[cid, pl.ds(j * ZROWS, ZROWS)])

    return k(y, src4d, dst4d, cf4d)


def kernel(features, edge_index, edgenet_input, pd_ftr_dim, nonimg, m, flag,
           pae_w1, pae_b1, pae_g, pae_beta, pae_w2, pae_b2,
           cheb_w0, cheb_w1, cheb_w2,
           cls_w1, cls_b1, cls_g, cls_beta, cls_w2, cls_b2):
    src2d = edge_index[0].astype(jnp.int32).reshape(NWORK, RW, CW)
    dst2d = edge_index[1].astype(jnp.int32).reshape(NWORK, RW, CW)

    b1r = pae_b1.reshape(1, 128)
    gr = pae_g.reshape(1, 128)
    betar = pae_beta.reshape(1, 128)
    b2r = pae_b2.reshape(1, 128)

    sums = _pae_sums(edgenet_input, pae_w1, b1r)
    ew = _pae_ew(edgenet_input, sums, pae_w1, b1r, gr, betar, pae_w2, b2r)
    ew2d = ew.reshape(NWORK, RW, CW)

    degp = _sc_deg(src2d, ew2d)
    dinv = _tc_dinv(degp.reshape(NWORK, NP))
    nw2d = _sc_nw(dinv.reshape(NP), src2d, dst2d, ew2d)

    src4d = src2d.reshape(NWORK, RW // RB, RB, CW)
    dst4d = dst2d.reshape(NWORK, RW // RB, RB, CW)
    nw4d = nw2d.reshape(NWORK, RW // RB, RB, CW)

    h = features
    hs = []
    for w in (cheb_w0, cheb_w1, cheb_w2):
        aparts = _sc_spmv(h, src4d, dst4d, nw4d)
        t1 = _tc_add(aparts)
        bparts = _sc_spmv(t1, src4d, dst4d, nw4d)
        h = _tc_layer(h, t1, bparts, w)
        hs.append(h)
    h0 = jnp.concatenate(hs, axis=1)

    cb1r = cls_b1.reshape(1, 256)
    cgr = cls_g.reshape(1, 256)
    cbetar = cls_beta.reshape(1, 256)
    w2p = jnp.zeros((256, 128), jnp.float32).at[:, :2].set(cls_w2)
    b2p = jnp.zeros((1, 128), jnp.float32).at[:, :2].set(cls_b2.reshape(1, 2))

    z, csums = _tc_cls1(h0, cls_w1, cb1r)
    logit_pad = _tc_cls2(z, csums, cgr, cbetar, w2p, b2p)
    logit = logit_pad[:, :2]
    return (h0, logit)


# speedup vs baseline: 9.6849x; 1.1877x over previous
import functools

import jax
import jax.numpy as jnp
from jax import lax
from jax.experimental import pallas as pl
from jax.experimental.pallas import tpu as pltpu
from jax.experimental.pallas import tpu_sc as plsc

N = 10000
E = 320000
D = 128
NP = 10240
CW = 80
NWORK = 32
RW = E // CW // NWORK
BE = 12800
BNODE = 1000
ZROWS = 400
RB = 25

def _mesh():
    return plsc.VectorSubcoreMesh(core_axis_name="c", subcore_axis_name="s",
                                  num_cores=2, num_subcores=16)


def _pae_sums_body(xt_ref, w1t_ref, b1c_ref, o_ref):
    @pl.when(pl.program_id(0) == 0)
    def _():
        o_ref[...] = jnp.zeros_like(o_ref)

    xt = xt_ref[...].astype(jnp.bfloat16)
    w1t = w1t_ref[...].astype(jnp.bfloat16)
    h1 = jax.nn.relu(jnp.dot(w1t, xt[:16],
                             preferred_element_type=jnp.float32) + b1c_ref[...])
    h2 = jax.nn.relu(jnp.dot(w1t, xt[16:],
                             preferred_element_type=jnp.float32) + b1c_ref[...])
    ones = jnp.ones((BE, 8), jnp.bfloat16)
    h1b = h1.astype(jnp.bfloat16)
    h2b = h2.astype(jnp.bfloat16)
    s1 = jnp.dot(h1b, ones, preferred_element_type=jnp.float32)
    q1 = jnp.dot(h1b * h1b, ones, preferred_element_type=jnp.float32)
    s2 = jnp.dot(h2b, ones, preferred_element_type=jnp.float32)
    q2 = jnp.dot(h2b * h2b, ones, preferred_element_type=jnp.float32)
    acc = jnp.concatenate([s1[:, 0:1], q1[:, 0:1], s2[:, 0:1], q2[:, 0:1],
                           jnp.zeros((128, 4), jnp.float32)], axis=1)
    o_ref[...] += acc


def _pae_sums(xt, w1t, b1c):
    return pl.pallas_call(
        _pae_sums_body,
        grid=(E // BE,),
        in_specs=[
            pl.BlockSpec((32, BE), lambda i: (0, i)),
            pl.BlockSpec((128, 16), lambda i: (0, 0)),
            pl.BlockSpec((128, 1), lambda i: (0, 0)),
        ],
        out_specs=pl.BlockSpec((128, 8), lambda i: (0, 0)),
        out_shape=jax.ShapeDtypeStruct((128, 8), jnp.float32),
    )(xt, w1t, b1c)


def _pae_ew_body(xt_ref, s_ref, w1t_ref, b1c_ref, g_ref, be_ref,
                 w2t_ref, b2c_ref, o_ref):
    inv_e = 1.0 / float(E)
    xt = xt_ref[...].astype(jnp.bfloat16)
    w1t = w1t_ref[...].astype(jnp.bfloat16)
    w2t = w2t_ref[...].astype(jnp.bfloat16)

    def branch(x, c0):
        mean = s_ref[:, c0:c0 + 1] * inv_e
        var = s_ref[:, c0 + 1:c0 + 2] * inv_e - mean * mean
        h = jax.nn.relu(jnp.dot(w1t, x,
                                preferred_element_type=jnp.float32) + b1c_ref[...])
        hn = (h - mean) * lax.rsqrt(var + 1e-5) * g_ref[...] + be_ref[...]
        return jnp.dot(w2t, hn.astype(jnp.bfloat16),
                       preferred_element_type=jnp.float32) + b2c_ref[...]

    g1 = branch(xt[:16], 0)
    g2 = branch(xt[16:], 2)
    n1 = jnp.maximum(jnp.sqrt(jnp.sum(g1 * g1, axis=0, keepdims=True)), 1e-8)
    n2 = jnp.maximum(jnp.sqrt(jnp.sum(g2 * g2, axis=0, keepdims=True)), 1e-8)
    cos = jnp.sum(g1 * g2, axis=0, keepdims=True) / (n1 * n2)
    o_ref[...] = jnp.broadcast_to((cos + 1.0) * 0.5, (8, BE))


def _pae_ew(xt, sums, w1t, b1c, gc, betac, w2t, b2c):
    return pl.pallas_call(
        _pae_ew_body,
        grid=(E // BE,),
        in_specs=[
            pl.BlockSpec((32, BE), lambda i: (0, i)),
            pl.BlockSpec((128, 8), lambda i: (0, 0)),
            pl.BlockSpec((128, 16), lambda i: (0, 0)),
            pl.BlockSpec((128, 1), lambda i: (0, 0)),
            pl.BlockSpec((128, 1), lambda i: (0, 0)),
            pl.BlockSpec((128, 1), lambda i: (0, 0)),
            pl.BlockSpec((128, 128), lambda i: (0, 0)),
            pl.BlockSpec((128, 1), lambda i: (0, 0)),
        ],
        out_specs=pl.BlockSpec((8, BE), lambda i: (0, i)),
        out_shape=jax.ShapeDtypeStruct((8, E), jnp.float32),
    )(xt, sums, w1t, b1c, gc, betac, w2t, b2c)


def _dinv_body(dp_ref, o_ref):
    deg = jnp.sum(dp_ref[...], axis=0, keepdims=True)
    dsafe = jnp.where(deg > 0, deg, 1.0)
    o_ref[...] = jnp.where(deg > 0, lax.rsqrt(dsafe), 0.0)


def _tc_dinv(degp):
    return pl.pallas_call(
        _dinv_body,
        out_shape=jax.ShapeDtypeStruct((1, NP), jnp.float32),
    )(degp)


def _addp_body(p_ref, o_ref):
    o_ref[...] = p_ref[0] + p_ref[1]


def _tc_add(parts):
    return pl.pallas_call(
        _addp_body,
        grid=(N // BNODE,),
        in_specs=[pl.BlockSpec((2, BNODE, D), lambda i: (0, i, 0))],
        out_specs=pl.BlockSpec((BNODE, D), lambda i: (i, 0)),
        out_shape=jax.ShapeDtypeStruct((N, D), jnp.float32),
    )(parts)


def _layer_body(y_ref, t1_ref, q_ref, w_ref, o_ref):
    t2 = 2.0 * (q_ref[0] + q_ref[1]) - y_ref[...]
    acc = jnp.dot(y_ref[...], w_ref[0], preferred_element_type=jnp.float32)
    acc += jnp.dot(t1_ref[...], w_ref[1], preferred_element_type=jnp.float32)
    acc += jnp.dot(t2, w_ref[2], preferred_element_type=jnp.float32)
    o_ref[...] = jax.nn.relu(acc)


def _tc_layer(y, t1, qparts, w):
    return pl.pallas_call(
        _layer_body,
        grid=(N // BNODE,),
        in_specs=[
            pl.BlockSpec((BNODE, D), lambda i: (i, 0)),
            pl.BlockSpec((BNODE, D), lambda i: (i, 0)),
            pl.BlockSpec((2, BNODE, D), lambda i: (0, i, 0)),
            pl.BlockSpec((3, D, D), lambda i: (0, 0, 0)),
        ],
        out_specs=pl.BlockSpec((BNODE, D), lambda i: (i, 0)),
        out_shape=jax.ShapeDtypeStruct((N, D), jnp.float32),
    )(y, t1, qparts, w)


def _cls1_body(h_ref, w1_ref, b1_ref, z_ref, s_ref):
    @pl.when(pl.program_id(0) == 0)
    def _():
        s_ref[...] = jnp.zeros_like(s_ref)

    z = jax.nn.relu(jnp.dot(h_ref[...], w1_ref[...],
                            preferred_element_type=jnp.float32) + b1_ref[...])
    z_ref[...] = z
    s_ref[...] += jnp.concatenate([
        jnp.sum(z, axis=0, keepdims=True),
        jnp.sum(z * z, axis=0, keepdims=True),
        jnp.zeros((6, 256), jnp.float32),
    ], axis=0)


def _tc_cls1(h0, w1, b1r):
    return pl.pallas_call(
        _cls1_body,
        grid=(N // BNODE,),
        in_specs=[
            pl.BlockSpec((BNODE, 384), lambda i: (i, 0)),
            pl.BlockSpec((384, 256), lambda i: (0, 0)),
            pl.BlockSpec((1, 256), lambda i: (0, 0)),
        ],
        out_specs=(
            pl.BlockSpec((BNODE, 256), lambda i: (i, 0)),
            pl.BlockSpec((8, 256), lambda i: (0, 0)),
        ),
        out_shape=(
            jax.ShapeDtypeStruct((N, 256), jnp.float32),
            jax.ShapeDtypeStruct((8, 256), jnp.float32),
        ),
    )(h0, w1, b1r)


def _cls2_body(z_ref, s_ref, g_ref, be_ref, w2_ref, b2_ref, o_ref):
    inv_n = 1.0 / float(N)
    mean = s_ref[0:1, :] * inv_n
    var = s_ref[1:2, :] * inv_n - mean * mean
    zn = (z_ref[...] - mean) * lax.rsqrt(var + 1e-5) * g_ref[...] + be_ref[...]
    o_ref[...] = jnp.dot(zn, w2_ref[...],
                         preferred_element_type=jnp.float32) + b2_ref[...]


def _tc_cls2(z, sums, gr, betar, w2p, b2p):
    return pl.pallas_call(
        _cls2_body,
        grid=(N // BNODE,),
        in_specs=[
            pl.BlockSpec((BNODE, 256), lambda i: (i, 0)),
            pl.BlockSpec((8, 256), lambda i: (0, 0)),
            pl.BlockSpec((1, 256), lambda i: (0, 0)),
            pl.BlockSpec((1, 256), lambda i: (0, 0)),
            pl.BlockSpec((256, 128), lambda i: (0, 0)),
            pl.BlockSpec((1, 128), lambda i: (0, 0)),
        ],
        out_specs=pl.BlockSpec((BNODE, 128), lambda i: (i, 0)),
        out_shape=jax.ShapeDtypeStruct((N, 128), jnp.float32),
    )(z, sums, gr, betar, w2p, b2p)


def _sc_deg(src2d, ew2d):
    @functools.partial(
        pl.kernel,
        out_type=jax.ShapeDtypeStruct((NWORK, 1, NP), jnp.float32),
        mesh=_mesh(),
        compiler_params=pltpu.CompilerParams(needs_layout_passes=False),
        scratch_types=[
            pltpu.VMEM((RW, CW), jnp.int32),
            pltpu.VMEM((RW, CW), jnp.float32),
            pltpu.VMEM((NP,), jnp.float32),
        ],
    )
    def k(src_hbm, ew_hbm, out_hbm, src_v, ew_v, acc_v):
        cid = lax.axis_index("c")
        sid = lax.axis_index("s")
        wid = cid * 16 + sid

        def zero(i, _):
            acc_v[pl.ds(i * 16, 16)] = jnp.zeros((16,), jnp.float32)
            return 0
        lax.fori_loop(0, NP // 16, zero, 0)

        pltpu.sync_copy(src_hbm.at[wid], src_v)
        pltpu.sync_copy(ew_hbm.at[wid], ew_v)

        def body(j, _):
            for g in range(CW // 16):
                idx = src_v[j, pl.ds(g * 16, 16)]
                val = ew_v[j, pl.ds(g * 16, 16)]
                plsc.addupdate_scatter(acc_v, [idx], val)
            return 0
        lax.fori_loop(0, RW, body, 0)

        pltpu.sync_copy(acc_v, out_hbm.at[wid, 0])

    return k(src2d, ew2d)


def _sc_nw(dinv1d, src2d, dst2d, ew2d):
    @functools.partial(
        pl.kernel,
        out_type=jax.ShapeDtypeStruct((NWORK, RW, CW), jnp.float32),
        mesh=_mesh(),
        compiler_params=pltpu.CompilerParams(needs_layout_passes=False),
        scratch_types=[
            pltpu.VMEM((NP,), jnp.float32),
            pltpu.VMEM((RW, CW), jnp.int32),
            pltpu.VMEM((RW, CW), jnp.int32),
            pltpu.VMEM((RW, CW), jnp.float32),
            pltpu.VMEM((RW, CW), jnp.float32),
        ],
    )
    def k(dinv_hbm, src_hbm, dst_hbm, ew_hbm, out_hbm,
          dinv_v, src_v, dst_v, ew_v, nw_v):
        cid = lax.axis_index("c")
        sid = lax.axis_index("s")
        wid = cid * 16 + sid

        pltpu.sync_copy(dinv_hbm, dinv_v)
        pltpu.sync_copy(src_hbm.at[wid], src_v)
        pltpu.sync_copy(dst_hbm.at[wid], dst_v)
        pltpu.sync_copy(ew_hbm.at[wid], ew_v)

        def body(j, _):
            for g in range(CW // 16):
                s16 = src_v[j, pl.ds(g * 16, 16)]
                d16 = dst_v[j, pl.ds(g * 16, 16)]
                w16 = ew_v[j, pl.ds(g * 16, 16)]
                ds_ = plsc.load_gather(dinv_v, [s16])
                dd_ = plsc.load_gather(dinv_v, [d16])
                nw_v[j, pl.ds(g * 16, 16)] = -(ds_ * w16 * dd_)
            return 0
        lax.fori_loop(0, RW, body, 0)

        pltpu.sync_copy(nw_v, out_hbm.at[wid])

    return k(dinv1d, src2d, dst2d, ew2d)


def _sc_spmv(y, src4d, dst4d, cf4d):
    @functools.partial(
        pl.kernel,
        out_type=jax.ShapeDtypeStruct((2, N, D), jnp.float32),
        mesh=_mesh(),
        compiler_params=pltpu.CompilerParams(needs_layout_passes=False),
        scratch_types=[
            pltpu.VMEM((RB, CW), jnp.int32),
            pltpu.VMEM((RB, CW), jnp.int32),
            pltpu.VMEM((RB, CW), jnp.float32),
            pltpu.VMEM((CW, D), jnp.float32),
            pltpu.VMEM((CW, D), jnp.float32),
            pltpu.VMEM_SHARED((N, D), jnp.float32),
            pltpu.SemaphoreType.DMA,
            pltpu.SemaphoreType.DMA,
            pltpu.SemaphoreType.DMA,
            pltpu.SemaphoreType.DMA,
        ],
    )
    def k(y_hbm, src_hbm, dst_hbm, cf_hbm, out_hbm,
          src_v, dst_v, cf_v, rows_a, rows_b, acc_s, gsa, gsb, ssa, ssb):
        cid = lax.axis_index("c")
        sid = lax.axis_index("s")
        wid = cid * 16 + sid

        def zrow(i, _):
            for l in range(D // 16):
                rows_a[i, pl.ds(l * 16, 16)] = jnp.zeros((16,), jnp.float32)
            return 0
        lax.fori_loop(0, CW, zrow, 0)

        def zchunk(j, _):
            @pl.when(sid == j % 16)
            def _():
                pltpu.sync_copy(rows_a, acc_s.at[pl.ds(j * CW, CW)])
            return 0
        lax.fori_loop(0, N // CW, zchunk, 0)
        plsc.subcore_barrier()

        def scale_rows(rows_x, jrow):
            def scale(g, _):
                cvec = cf_v[jrow, pl.ds(g * 16, 16)]
                for e16 in range(16):
                    c = cvec[e16]
                    e = g * 16 + e16
                    for l in range(D // 16):
                        rows_x[e, pl.ds(l * 16, 16)] = (
                            rows_x[e, pl.ds(l * 16, 16)] * c)
                return 0
            lax.fori_loop(0, CW // 16, scale, 0)

        def gwait(rows_x, jrow, sem_x):
            pltpu.make_async_copy(y_hbm.at[src_v.at[jrow]], rows_x, sem_x).wait()

        def swait(rows_x, jrow, sem_x):
            pltpu.make_async_copy(rows_x, acc_s.at[dst_v.at[jrow]], sem_x).wait()

        def batch(b, _):
            @pl.when(b >= 1)
            def _():
                swait(rows_a, RB - 1, ssa)
                swait(rows_b, RB - 2, ssb)
            pltpu.sync_copy(src_hbm.at[wid, b], src_v)
            pltpu.sync_copy(dst_hbm.at[wid, b], dst_v)
            pltpu.sync_copy(cf_hbm.at[wid, b], cf_v)
            pltpu.async_copy(y_hbm.at[src_v.at[0]], rows_a, gsa)

            def pair(m2, _):
                j0 = 2 * m2
                j1 = j0 + 1
                @pl.when(m2 >= 1)
                def _():
                    swait(rows_b, j0 - 1, ssb)
                pltpu.async_copy(y_hbm.at[src_v.at[j1]], rows_b, gsb)
                gwait(rows_a, j0, gsa)
                scale_rows(rows_a, j0)
                pltpu.async_copy(rows_a, acc_s.at[dst_v.at[j0]], ssa, add=True)
                swait(rows_a, j0, ssa)
                pltpu.async_copy(y_hbm.at[src_v.at[j1 + 1]], rows_a, gsa)
                gwait(rows_b, j1, gsb)
                scale_rows(rows_b, j1)
                pltpu.async_copy(rows_b, acc_s.at[dst_v.at[j1]], ssb, add=True)
                return 0
            lax.fori_loop(0, RB // 2, pair, 0)
            gwait(rows_a, RB - 1, gsa)
            scale_rows(rows_a, RB - 1)
            pltpu.async_copy(rows_a, acc_s.at[dst_v.at[RB - 1]], ssa, add=True)
            return 0
        lax.fori_loop(0, RW // RB, batch, 0)

        swait(rows_a, RB - 1, ssa)
        swait(rows_b, RB - 2, ssb)
        plsc.subcore_barrier()
        for j in range(N // ZROWS):
            @pl.when(sid == j % 16)
            def _():
                pltpu.sync_copy(acc_s.at[pl.ds(j * ZROWS, ZROWS)],
                                out_hbm.at[cid, pl.ds(j * ZROWS, ZROWS)])

    return k(y, src4d, dst4d, cf4d)


def kernel(features, edge_index, edgenet_input, pd_ftr_dim, nonimg, m, flag,
           pae_w1, pae_b1, pae_g, pae_beta, pae_w2, pae_b2,
           cheb_w0, cheb_w1, cheb_w2,
           cls_w1, cls_b1, cls_g, cls_beta, cls_w2, cls_b2):
    src2d = edge_index[0].astype(jnp.int32).reshape(NWORK, RW, CW)
    dst2d = edge_index[1].astype(jnp.int32).reshape(NWORK, RW, CW)

    xt = edgenet_input.T
    w1t = pae_w1.T
    w2t = pae_w2.T
    b1c = pae_b1.reshape(128, 1)
    gc = pae_g.reshape(128, 1)
    betac = pae_beta.reshape(128, 1)
    b2c = pae_b2.reshape(128, 1)

    sums = _pae_sums(xt, w1t, b1c)
    ew8 = _pae_ew(xt, sums, w1t, b1c, gc, betac, w2t, b2c)
    ew2d = ew8[0].reshape(NWORK, RW, CW)

    degp = _sc_deg(src2d, ew2d)
    dinv = _tc_dinv(degp.reshape(NWORK, NP))
    nw2d = _sc_nw(dinv.reshape(NP), src2d, dst2d, ew2d)

    src4d = src2d.reshape(NWORK, RW // RB, RB, CW)
    dst4d = dst2d.reshape(NWORK, RW // RB, RB, CW)
    nw4d = nw2d.reshape(NWORK, RW // RB, RB, CW)

    h = features
    hs = []
    for w in (cheb_w0, cheb_w1, cheb_w2):
        aparts = _sc_spmv(h, src4d, dst4d, nw4d)
        t1 = _tc_add(aparts)
        bparts = _sc_spmv(t1, src4d, dst4d, nw4d)
        h = _tc_layer(h, t1, bparts, w)
        hs.append(h)
    h0 = jnp.concatenate(hs, axis=1)

    cb1r = cls_b1.reshape(1, 256)
    cgr = cls_g.reshape(1, 256)
    cbetar = cls_beta.reshape(1, 256)
    w2p = jnp.zeros((256, 128), jnp.float32).at[:, :2].set(cls_w2)
    b2p = jnp.zeros((1, 128), jnp.float32).at[:, :2].set(cls_b2.reshape(1, 2))

    z, csums = _tc_cls1(h0, cls_w1, cb1r)
    logit_pad = _tc_cls2(z, csums, cgr, cbetar, w2p, b2p)
    logit = logit_pad[:, :2]
    return (h0, logit)


# speedup vs baseline: 9.7868x; 1.0105x over previous
import functools

import jax
import jax.numpy as jnp
from jax import lax
from jax.experimental import pallas as pl
from jax.experimental.pallas import tpu as pltpu
from jax.experimental.pallas import tpu_sc as plsc

N = 10000
E = 320000
D = 128
NP = 10240
CW = 80
NWORK = 32
RW = E // CW // NWORK
BE = 12800
BNODE = 1000
ZROWS = 400
RB = 25

def _mesh():
    return plsc.VectorSubcoreMesh(core_axis_name="c", subcore_axis_name="s",
                                  num_cores=2, num_subcores=16)


def _pae_sums_body(xt_ref, w1t_ref, b1c_ref, o_ref):
    @pl.when(pl.program_id(0) == 0)
    def _():
        o_ref[...] = jnp.zeros_like(o_ref)

    xt = xt_ref[...].astype(jnp.bfloat16)
    w1t = w1t_ref[...].astype(jnp.bfloat16)
    h1 = jax.nn.relu(jnp.dot(w1t, xt[:16],
                             preferred_element_type=jnp.float32) + b1c_ref[...])
    h2 = jax.nn.relu(jnp.dot(w1t, xt[16:],
                             preferred_element_type=jnp.float32) + b1c_ref[...])
    ones = jnp.ones((BE, 128), jnp.bfloat16)
    h1b = h1.astype(jnp.bfloat16)
    h2b = h2.astype(jnp.bfloat16)
    s1 = jnp.dot(h1b, ones, preferred_element_type=jnp.float32)
    q1 = jnp.dot(h1b * h1b, ones, preferred_element_type=jnp.float32)
    s2 = jnp.dot(h2b, ones, preferred_element_type=jnp.float32)
    q2 = jnp.dot(h2b * h2b, ones, preferred_element_type=jnp.float32)
    acc = jnp.concatenate([s1[:, 0:1], q1[:, 0:1], s2[:, 0:1], q2[:, 0:1],
                           jnp.zeros((128, 4), jnp.float32)], axis=1)
    o_ref[...] += acc


def _pae_sums(xt, w1t, b1c):
    return pl.pallas_call(
        _pae_sums_body,
        grid=(E // BE,),
        in_specs=[
            pl.BlockSpec((32, BE), lambda i: (0, i)),
            pl.BlockSpec((128, 16), lambda i: (0, 0)),
            pl.BlockSpec((128, 1), lambda i: (0, 0)),
        ],
        out_specs=pl.BlockSpec((128, 8), lambda i: (0, 0)),
        out_shape=jax.ShapeDtypeStruct((128, 8), jnp.float32),
    )(xt, w1t, b1c)


def _pae_ew_body(xt_ref, s_ref, w1t_ref, b1c_ref, g_ref, be_ref,
                 w2t_ref, b2c_ref, o_ref):
    inv_e = 1.0 / float(E)
    xt = xt_ref[...].astype(jnp.bfloat16)
    w1t = w1t_ref[...].astype(jnp.bfloat16)
    w2t = w2t_ref[...].astype(jnp.bfloat16)

    def branch(x, c0):
        mean = s_ref[:, c0:c0 + 1] * inv_e
        var = s_ref[:, c0 + 1:c0 + 2] * inv_e - mean * mean
        h = jax.nn.relu(jnp.dot(w1t, x,
                                preferred_element_type=jnp.float32) + b1c_ref[...])
        hn = (h - mean) * lax.rsqrt(var + 1e-5) * g_ref[...] + be_ref[...]
        return jnp.dot(w2t, hn.astype(jnp.bfloat16),
                       preferred_element_type=jnp.float32) + b2c_ref[...]

    g1 = branch(xt[:16], 0)
    g2 = branch(xt[16:], 2)
    ones8 = jnp.ones((8, 128), jnp.bfloat16)
    g1b = g1.astype(jnp.bfloat16)
    g2b = g2.astype(jnp.bfloat16)
    nn1 = jnp.dot(ones8, g1b * g1b, preferred_element_type=jnp.float32)
    nn2 = jnp.dot(ones8, g2b * g2b, preferred_element_type=jnp.float32)
    dd = jnp.dot(ones8, g1b * g2b, preferred_element_type=jnp.float32)
    n1 = jnp.maximum(jnp.sqrt(nn1), 1e-8)
    n2 = jnp.maximum(jnp.sqrt(nn2), 1e-8)
    cos = dd / (n1 * n2)
    o_ref[...] = (cos + 1.0) * 0.5


def _pae_ew(xt, sums, w1t, b1c, gc, betac, w2t, b2c):
    return pl.pallas_call(
        _pae_ew_body,
        grid=(E // BE,),
        in_specs=[
            pl.BlockSpec((32, BE), lambda i: (0, i)),
            pl.BlockSpec((128, 8), lambda i: (0, 0)),
            pl.BlockSpec((128, 16), lambda i: (0, 0)),
            pl.BlockSpec((128, 1), lambda i: (0, 0)),
            pl.BlockSpec((128, 1), lambda i: (0, 0)),
            pl.BlockSpec((128, 1), lambda i: (0, 0)),
            pl.BlockSpec((128, 128), lambda i: (0, 0)),
            pl.BlockSpec((128, 1), lambda i: (0, 0)),
        ],
        out_specs=pl.BlockSpec((8, BE), lambda i: (0, i)),
        out_shape=jax.ShapeDtypeStruct((8, E), jnp.float32),
    )(xt, sums, w1t, b1c, gc, betac, w2t, b2c)


def _dinv_body(dp_ref, o_ref):
    deg = jnp.sum(dp_ref[...], axis=0, keepdims=True)
    dsafe = jnp.where(deg > 0, deg, 1.0)
    o_ref[...] = jnp.where(deg > 0, lax.rsqrt(dsafe), 0.0)


def _tc_dinv(degp):
    return pl.pallas_call(
        _dinv_body,
        out_shape=jax.ShapeDtypeStruct((1, NP), jnp.float32),
    )(degp)


def _addp_body(p_ref, o_ref):
    o_ref[...] = p_ref[0] + p_ref[1]


def _tc_add(parts):
    return pl.pallas_call(
        _addp_body,
        grid=(N // BNODE,),
        in_specs=[pl.BlockSpec((2, BNODE, D), lambda i: (0, i, 0))],
        out_specs=pl.BlockSpec((BNODE, D), lambda i: (i, 0)),
        out_shape=jax.ShapeDtypeStruct((N, D), jnp.float32),
    )(parts)


def _layer_body(y_ref, t1_ref, q_ref, w_ref, o_ref):
    t2 = 2.0 * (q_ref[0] + q_ref[1]) - y_ref[...]
    acc = jnp.dot(y_ref[...], w_ref[0], preferred_element_type=jnp.float32)
    acc += jnp.dot(t1_ref[...], w_ref[1], preferred_element_type=jnp.float32)
    acc += jnp.dot(t2, w_ref[2], preferred_element_type=jnp.float32)
    o_ref[...] = jax.nn.relu(acc)


def _tc_layer(y, t1, qparts, w):
    return pl.pallas_call(
        _layer_body,
        grid=(N // BNODE,),
        in_specs=[
            pl.BlockSpec((BNODE, D), lambda i: (i, 0)),
            pl.BlockSpec((BNODE, D), lambda i: (i, 0)),
            pl.BlockSpec((2, BNODE, D), lambda i: (0, i, 0)),
            pl.BlockSpec((3, D, D), lambda i: (0, 0, 0)),
        ],
        out_specs=pl.BlockSpec((BNODE, D), lambda i: (i, 0)),
        out_shape=jax.ShapeDtypeStruct((N, D), jnp.float32),
    )(y, t1, qparts, w)


def _cls1_body(h_ref, w1_ref, b1_ref, z_ref, s_ref):
    @pl.when(pl.program_id(0) == 0)
    def _():
        s_ref[...] = jnp.zeros_like(s_ref)

    z = jax.nn.relu(jnp.dot(h_ref[...], w1_ref[...],
                            preferred_element_type=jnp.float32) + b1_ref[...])
    z_ref[...] = z
    s_ref[...] += jnp.concatenate([
        jnp.sum(z, axis=0, keepdims=True),
        jnp.sum(z * z, axis=0, keepdims=True),
        jnp.zeros((6, 256), jnp.float32),
    ], axis=0)


def _tc_cls1(h0, w1, b1r):
    return pl.pallas_call(
        _cls1_body,
        grid=(N // BNODE,),
        in_specs=[
            pl.BlockSpec((BNODE, 384), lambda i: (i, 0)),
            pl.BlockSpec((384, 256), lambda i: (0, 0)),
            pl.BlockSpec((1, 256), lambda i: (0, 0)),
        ],
        out_specs=(
            pl.BlockSpec((BNODE, 256), lambda i: (i, 0)),
            pl.BlockSpec((8, 256), lambda i: (0, 0)),
        ),
        out_shape=(
            jax.ShapeDtypeStruct((N, 256), jnp.float32),
            jax.ShapeDtypeStruct((8, 256), jnp.float32),
        ),
    )(h0, w1, b1r)


def _cls2_body(z_ref, s_ref, g_ref, be_ref, w2_ref, b2_ref, o_ref):
    inv_n = 1.0 / float(N)
    mean = s_ref[0:1, :] * inv_n
    var = s_ref[1:2, :] * inv_n - mean * mean
    zn = (z_ref[...] - mean) * lax.rsqrt(var + 1e-5) * g_ref[...] + be_ref[...]
    o_ref[...] = jnp.dot(zn, w2_ref[...],
                         preferred_element_type=jnp.float32) + b2_ref[...]


def _tc_cls2(z, sums, gr, betar, w2p, b2p):
    return pl.pallas_call(
        _cls2_body,
        grid=(N // BNODE,),
        in_specs=[
            pl.BlockSpec((BNODE, 256), lambda i: (i, 0)),
            pl.BlockSpec((8, 256), lambda i: (0, 0)),
            pl.BlockSpec((1, 256), lambda i: (0, 0)),
            pl.BlockSpec((1, 256), lambda i: (0, 0)),
            pl.BlockSpec((256, 128), lambda i: (0, 0)),
            pl.BlockSpec((1, 128), lambda i: (0, 0)),
        ],
        out_specs=pl.BlockSpec((BNODE, 128), lambda i: (i, 0)),
        out_shape=jax.ShapeDtypeStruct((N, 128), jnp.float32),
    )(z, sums, gr, betar, w2p, b2p)


def _sc_deg(src2d, ew2d):
    @functools.partial(
        pl.kernel,
        out_type=jax.ShapeDtypeStruct((NWORK, 1, NP), jnp.float32),
        mesh=_mesh(),
        compiler_params=pltpu.CompilerParams(needs_layout_passes=False),
        scratch_types=[
            pltpu.VMEM((RW, CW), jnp.int32),
            pltpu.VMEM((RW, CW), jnp.float32),
            pltpu.VMEM((NP,), jnp.float32),
        ],
    )
    def k(src_hbm, ew_hbm, out_hbm, src_v, ew_v, acc_v):
        cid = lax.axis_index("c")
        sid = lax.axis_index("s")
        wid = cid * 16 + sid

        def zero(i, _):
            acc_v[pl.ds(i * 16, 16)] = jnp.zeros((16,), jnp.float32)
            return 0
        lax.fori_loop(0, NP // 16, zero, 0)

        pltpu.sync_copy(src_hbm.at[wid], src_v)
        pltpu.sync_copy(ew_hbm.at[wid], ew_v)

        def body(j, _):
            for g in range(CW // 16):
                idx = src_v[j, pl.ds(g * 16, 16)]
                val = ew_v[j, pl.ds(g * 16, 16)]
                plsc.addupdate_scatter(acc_v, [idx], val)
            return 0
        lax.fori_loop(0, RW, body, 0)

        pltpu.sync_copy(acc_v, out_hbm.at[wid, 0])

    return k(src2d, ew2d)


def _sc_nw(dinv1d, src2d, dst2d, ew2d):
    @functools.partial(
        pl.kernel,
        out_type=jax.ShapeDtypeStruct((NWORK, RW, CW), jnp.float32),
        mesh=_mesh(),
        compiler_params=pltpu.CompilerParams(needs_layout_passes=False),
        scratch_types=[
            pltpu.VMEM((NP,), jnp.float32),
            pltpu.VMEM((RW, CW), jnp.int32),
            pltpu.VMEM((RW, CW), jnp.int32),
            pltpu.VMEM((RW, CW), jnp.float32),
            pltpu.VMEM((RW, CW), jnp.float32),
        ],
    )
    def k(dinv_hbm, src_hbm, dst_hbm, ew_hbm, out_hbm,
          dinv_v, src_v, dst_v, ew_v, nw_v):
        cid = lax.axis_index("c")
        sid = lax.axis_index("s")
        wid = cid * 16 + sid

        pltpu.sync_copy(dinv_hbm, dinv_v)
        pltpu.sync_copy(src_hbm.at[wid], src_v)
        pltpu.sync_copy(dst_hbm.at[wid], dst_v)
        pltpu.sync_copy(ew_hbm.at[wid], ew_v)

        def body(j, _):
            for g in range(CW // 16):
                s16 = src_v[j, pl.ds(g * 16, 16)]
                d16 = dst_v[j, pl.ds(g * 16, 16)]
                w16 = ew_v[j, pl.ds(g * 16, 16)]
                ds_ = plsc.load_gather(dinv_v, [s16])
                dd_ = plsc.load_gather(dinv_v, [d16])
                nw_v[j, pl.ds(g * 16, 16)] = -(ds_ * w16 * dd_)
            return 0
        lax.fori_loop(0, RW, body, 0)

        pltpu.sync_copy(nw_v, out_hbm.at[wid])

    return k(dinv1d, src2d, dst2d, ew2d)


def _sc_spmv(y, src4d, dst4d, cf4d):
    @functools.partial(
        pl.kernel,
        out_type=jax.ShapeDtypeStruct((2, N, D), jnp.float32),
        mesh=_mesh(),
        compiler_params=pltpu.CompilerParams(needs_layout_passes=False),
        scratch_types=[
            pltpu.VMEM((RB, CW), jnp.int32),
            pltpu.VMEM((RB, CW), jnp.int32),
            pltpu.VMEM((RB, CW), jnp.float32),
            pltpu.VMEM((CW, D), jnp.float32),
            pltpu.VMEM((CW, D), jnp.float32),
            pltpu.VMEM_SHARED((N, D), jnp.float32),
            pltpu.SemaphoreType.DMA,
            pltpu.SemaphoreType.DMA,
            pltpu.SemaphoreType.DMA,
            pltpu.SemaphoreType.DMA,
        ],
    )
    def k(y_hbm, src_hbm, dst_hbm, cf_hbm, out_hbm,
          src_v, dst_v, cf_v, rows_a, rows_b, acc_s, gsa, gsb, ssa, ssb):
        cid = lax.axis_index("c")
        sid = lax.axis_index("s")
        wid = cid * 16 + sid

        def zrow(i, _):
            for l in range(D // 16):
                rows_a[i, pl.ds(l * 16, 16)] = jnp.zeros((16,), jnp.float32)
            return 0
        lax.fori_loop(0, CW, zrow, 0)

        def zchunk(j, _):
            @pl.when(sid == j % 16)
            def _():
                pltpu.sync_copy(rows_a, acc_s.at[pl.ds(j * CW, CW)])
            return 0
        lax.fori_loop(0, N // CW, zchunk, 0)
        plsc.subcore_barrier()

        def scale_rows(rows_x, jrow):
            def scale(g, _):
                cvec = cf_v[jrow, pl.ds(g * 16, 16)]
                for e16 in range(16):
                    c = cvec[e16]
                    e = g * 16 + e16
                    for l in range(D // 16):
                        rows_x[e, pl.ds(l * 16, 16)] = (
                            rows_x[e, pl.ds(l * 16, 16)] * c)
                return 0
            lax.fori_loop(0, CW // 16, scale, 0)

        def gwait(rows_x, jrow, sem_x):
            pltpu.make_async_copy(y_hbm.at[src_v.at[jrow]], rows_x, sem_x).wait()

        def swait(rows_x, jrow, sem_x):
            pltpu.make_async_copy(rows_x, acc_s.at[dst_v.at[jrow]], sem_x).wait()

        def batch(b, _):
            @pl.when(b >= 1)
            def _():
                swait(rows_a, RB - 1, ssa)
                swait(rows_b, RB - 2, ssb)
            pltpu.sync_copy(src_hbm.at[wid, b], src_v)
            pltpu.sync_copy(dst_hbm.at[wid, b], dst_v)
            pltpu.sync_copy(cf_hbm.at[wid, b], cf_v)
            pltpu.async_copy(y_hbm.at[src_v.at[0]], rows_a, gsa)

            def pair(m2, _):
                j0 = 2 * m2
                j1 = j0 + 1
                @pl.when(m2 >= 1)
                def _():
                    swait(rows_b, j0 - 1, ssb)
                pltpu.async_copy(y_hbm.at[src_v.at[j1]], rows_b, gsb)
                gwait(rows_a, j0, gsa)
                scale_rows(rows_a, j0)
                pltpu.async_copy(rows_a, acc_s.at[dst_v.at[j0]], ssa, add=True)
                swait(rows_a, j0, ssa)
                pltpu.async_copy(y_hbm.at[src_v.at[j1 + 1]], rows_a, gsa)
                gwait(rows_b, j1, gsb)
                scale_rows(rows_b, j1)
                pltpu.async_copy(rows_b, acc_s.at[dst_v.at[j1]], ssb, add=True)
                return 0
            lax.fori_loop(0, RB // 2, pair, 0)
            gwait(rows_a, RB - 1, gsa)
            scale_rows(rows_a, RB - 1)
            pltpu.async_copy(rows_a, acc_s.at[dst_v.at[RB - 1]], ssa, add=True)
            return 0
        lax.fori_loop(0, RW // RB, batch, 0)

        swait(rows_a, RB - 1, ssa)
        swait(rows_b, RB - 2, ssb)
        plsc.subcore_barrier()
        for j in range(N // ZROWS):
            @pl.when(sid == j % 16)
            def _():
                pltpu.sync_copy(acc_s.at[pl.ds(j * ZROWS, ZROWS)],
                                out_hbm.at[cid, pl.ds(j * ZROWS, ZROWS)])

    return k(y, src4d, dst4d, cf4d)


def kernel(features, edge_index, edgenet_input, pd_ftr_dim, nonimg, m, flag,
           pae_w1, pae_b1, pae_g, pae_beta, pae_w2, pae_b2,
           cheb_w0, cheb_w1, cheb_w2,
           cls_w1, cls_b1, cls_g, cls_beta, cls_w2, cls_b2):
    src2d = edge_index[0].astype(jnp.int32).reshape(NWORK, RW, CW)
    dst2d = edge_index[1].astype(jnp.int32).reshape(NWORK, RW, CW)

    xt = edgenet_input.T
    w1t = pae_w1.T
    w2t = pae_w2.T
    b1c = pae_b1.reshape(128, 1)
    gc = pae_g.reshape(128, 1)
    betac = pae_beta.reshape(128, 1)
    b2c = pae_b2.reshape(128, 1)

    sums = _pae_sums(xt, w1t, b1c)
    ew8 = _pae_ew(xt, sums, w1t, b1c, gc, betac, w2t, b2c)
    ew2d = ew8[0].reshape(NWORK, RW, CW)

    degp = _sc_deg(src2d, ew2d)
    dinv = _tc_dinv(degp.reshape(NWORK, NP))
    nw2d = _sc_nw(dinv.reshape(NP), src2d, dst2d, ew2d)

    src4d = src2d.reshape(NWORK, RW // RB, RB, CW)
    dst4d = dst2d.reshape(NWORK, RW // RB, RB, CW)
    nw4d = nw2d.reshape(NWORK, RW // RB, RB, CW)

    h = features
    hs = []
    for w in (cheb_w0, cheb_w1, cheb_w2):
        aparts = _sc_spmv(h, src4d, dst4d, nw4d)
        t1 = _tc_add(aparts)
        bparts = _sc_spmv(t1, src4d, dst4d, nw4d)
        h = _tc_layer(h, t1, bparts, w)
        hs.append(h)
    h0 = jnp.concatenate(hs, axis=1)

    cb1r = cls_b1.reshape(1, 256)
    cgr = cls_g.reshape(1, 256)
    cbetar = cls_beta.reshape(1, 256)
    w2p = jnp.zeros((256, 128), jnp.float32).at[:, :2].set(cls_w2)
    b2p = jnp.zeros((1, 128), jnp.float32).at[:, :2].set(cls_b2.reshape(1, 2))

    z, csums = _tc_cls1(h0, cls_w1, cb1r)
    logit_pad = _tc_cls2(z, csums, cgr, cbetar, w2p, b2p)
    logit = logit_pad[:, :2]
    return (h0, logit)
